# baseline (speedup 1.0000x reference)
import functools

import jax
import jax.numpy as jnp
from jax import lax
from jax.experimental import pallas as pl
from jax.experimental.pallas import tpu as pltpu
from jax.experimental.pallas import tpu_sc as plsc

_NC = 2
_NS = 16
_NW = _NC * _NS
_CH = 8
_NB = 8


def _sc_add_kernel(x_hbm, emb_hbm, out_hbm, bufs):
    B = x_hbm.shape[0]
    T = x_hbm.shape[1]
    D = x_hbm.shape[2]
    tpw = T // _NW
    nch = tpw // _CH
    steps = nch * B
    ngrp = steps // _NB
    cpg = _NB // B
    wid = lax.axis_index("s") * _NC + lax.axis_index("c")
    t0 = wid * tpw
    xb = bufs[:_NB]
    embb = bufs[_NB:_NB + cpg]
    sin = bufs[_NB + cpg:2 * _NB + cpg]
    sout = bufs[2 * _NB + cpg:3 * _NB + cpg]
    semb = bufs[3 * _NB + cpg:]

    def in_copy(cc, b, k):
        return pltpu.make_async_copy(
            x_hbm.at[b, pl.ds(t0 + cc * _CH, _CH)], xb[k], sin[k])

    def out_copy(cc, b, k):
        return pltpu.make_async_copy(
            xb[k], out_hbm.at[b, pl.ds(t0 + cc * _CH, _CH)], sout[k])

    def emb_copy(cc, j):
        return pltpu.make_async_copy(
            emb_hbm.at[pl.ds(t0 + cc * _CH, _CH)], embb[j], semb[j])

    for j in range(cpg):
        emb_copy(j, j).start()
    for k in range(_NB):
        in_copy(k // B, k % B, k).start()

    def group(g, carry):
        cc0 = g * cpg

        def refill(k2):
            j2, b2 = k2 // B, k2 % B
            cc2 = cc0 + j2

            @pl.when(g < ngrp - 1)
            def _():
                out_copy(cc2, b2, k2).wait()
                in_copy(cc2 + cpg, b2, k2).start()

        for k in range(_NB):
            j = k // B
            b = k % B
            cc = cc0 + j
            if b == 0:
                emb_copy(cc, j).wait()
            in_copy(cc, b, k).wait()
            buf = xb[k]
            emb_buf = embb[j]

            @plsc.parallel_loop(0, _CH, step=1, unroll=2)
            def _row(r):
                for c in range(D // 16):
                    v = emb_buf[r, pl.ds(c * 16, 16)]
                    plsc.addupdate(buf.at[r, pl.ds(c * 16, 16)], v)

            out_copy(cc, b, k).start()
            if b == B - 1:
                @pl.when(g < ngrp - 1)
                def _():
                    emb_copy(cc + cpg, j).start()
            if k >= 2:
                refill(k - 2)
        refill(_NB - 2)
        refill(_NB - 1)
        return carry

    lax.fori_loop(0, ngrp, group, 0)
    for k in range(_NB):
        out_copy(nch - cpg + k // B, k % B, k).wait()


def kernel(x, emb):
    B, T, D = x.shape
    cpg = _NB // B
    sc_call = functools.partial(
        pl.kernel,
        out_type=jax.ShapeDtypeStruct((B, T, D), x.dtype),
        mesh=plsc.VectorSubcoreMesh(core_axis_name="c", subcore_axis_name="s"),
        scratch_types=[
            [pltpu.VMEM((_CH, D), jnp.float32)] * (_NB + cpg)
            + [pltpu.SemaphoreType.DMA] * (2 * _NB + cpg),
        ],
    )(_sc_add_kernel)
    return sc_call(x, emb[:T])


# speedup vs baseline: 1.0281x; 1.0281x over previous
import functools

import jax
import jax.numpy as jnp
from jax import lax
from jax.experimental import pallas as pl
from jax.experimental.pallas import tpu as pltpu
from jax.experimental.pallas import tpu_sc as plsc

_NC = 2
_NS = 16
_NW = _NC * _NS
_CH = 16
_NB = 4


def _sc_add_kernel(x_hbm, emb_hbm, out_hbm, bufs):
    B = x_hbm.shape[0]
    T = x_hbm.shape[1]
    D = x_hbm.shape[2]
    tpw = T // _NW
    nch = tpw // _CH
    steps = nch * B
    ngrp = steps // _NB
    cpg = _NB // B
    wid = lax.axis_index("s") * _NC + lax.axis_index("c")
    t0 = wid * tpw
    xb = bufs[:_NB]
    embb = bufs[_NB:_NB + cpg]
    sin = bufs[_NB + cpg:2 * _NB + cpg]
    sout = bufs[2 * _NB + cpg:3 * _NB + cpg]
    semb = bufs[3 * _NB + cpg:]

    def in_copy(cc, b, k):
        return pltpu.make_async_copy(
            x_hbm.at[b, pl.ds(t0 + cc * _CH, _CH)], xb[k], sin[k])

    def out_copy(cc, b, k):
        return pltpu.make_async_copy(
            xb[k], out_hbm.at[b, pl.ds(t0 + cc * _CH, _CH)], sout[k])

    def emb_copy(cc, j):
        return pltpu.make_async_copy(
            emb_hbm.at[pl.ds(t0 + cc * _CH, _CH)], embb[j], semb[j])

    for j in range(cpg):
        emb_copy(j, j).start()
    for k in range(_NB):
        in_copy(k // B, k % B, k).start()

    def group(g, carry):
        cc0 = g * cpg

        def refill(k2):
            j2, b2 = k2 // B, k2 % B
            cc2 = cc0 + j2

            @pl.when(g < ngrp - 1)
            def _():
                out_copy(cc2, b2, k2).wait()
                in_copy(cc2 + cpg, b2, k2).start()

        for k in range(_NB):
            j = k // B
            b = k % B
            cc = cc0 + j
            if b == 0:
                emb_copy(cc, j).wait()
            in_copy(cc, b, k).wait()
            buf = xb[k]
            emb_buf = embb[j]

            @plsc.parallel_loop(0, _CH, step=1, unroll=2)
            def _row(r):
                for c in range(D // 16):
                    v = emb_buf[r, pl.ds(c * 16, 16)]
                    plsc.addupdate(buf.at[r, pl.ds(c * 16, 16)], v)

            out_copy(cc, b, k).start()
            if b == B - 1:
                @pl.when(g < ngrp - 1)
                def _():
                    emb_copy(cc + cpg, j).start()
            if k >= 2:
                refill(k - 2)
        refill(_NB - 2)
        refill(_NB - 1)
        return carry

    lax.fori_loop(0, ngrp, group, 0)
    for k in range(_NB):
        out_copy(nch - cpg + k // B, k % B, k).wait()


def kernel(x, emb):
    B, T, D = x.shape
    cpg = _NB // B
    sc_call = functools.partial(
        pl.kernel,
        out_type=jax.ShapeDtypeStruct((B, T, D), x.dtype),
        mesh=plsc.VectorSubcoreMesh(core_axis_name="c", subcore_axis_name="s"),
        scratch_types=[
            [pltpu.VMEM((_CH, D), jnp.float32)] * (_NB + cpg)
            + [pltpu.SemaphoreType.DMA] * (2 * _NB + cpg),
        ],
    )(_sc_add_kernel)
    return sc_call(x, emb[:T])


# speedup vs baseline: 3.0144x; 2.9321x over previous
import jax
from jax.experimental import pallas as pl


def _add_kernel(x_ref, emb_ref, o_ref):
    o_ref[...] = x_ref[...] + emb_ref[...]


def kernel(x, emb):
    B, T, D = x.shape
    TB = 256
    return pl.pallas_call(
        _add_kernel,
        grid=(T // TB,),
        in_specs=[
            pl.BlockSpec((B, TB, D), lambda i: (0, i, 0)),
            pl.BlockSpec((TB, D), lambda i: (i, 0)),
        ],
        out_specs=pl.BlockSpec((B, TB, D), lambda i: (0, i, 0)),
        out_shape=jax.ShapeDtypeStruct((B, T, D), x.dtype),
    )(x, emb)
